# Initial kernel scaffold; baseline (speedup 1.0000x reference)
import jax
import jax.numpy as jnp
from jax.experimental import pallas as pl


def kernel(log_probs, targets, input_lengths, target_lengths):
    raise NotImplementedError("write your pallas kernel here")


# speedup vs baseline: 53.2811x; 53.2811x over previous
import functools

import jax
import jax.numpy as jnp
from jax.experimental import pallas as pl
from jax.experimental.pallas import tpu as pltpu

NEGBIG = -1e30


def _gather_kernel(lp_ref, cls_ref, w_ref):
    x = lp_ref[0]
    m = jnp.max(x, axis=1, keepdims=True)
    z = jnp.sum(jnp.exp(x - m), axis=1, keepdims=True)
    lse = m + jnp.log(z)
    C = x.shape[1]
    cls = cls_ref[0]
    cidx = jax.lax.broadcasted_iota(jnp.int32, (C, 128), 0)
    oh = (cidx == cls).astype(jnp.float32)
    g = jnp.dot(x, oh, preferred_element_type=jnp.float32)
    w_ref[0] = g - lse


def _rot64(x):
    return jnp.concatenate([x[:, 64:], x[:, :64]], axis=1)


def _shift1(x, fill):
    b = x.shape[0]
    return jnp.concatenate([jnp.full((b, 1), fill, x.dtype), x[:, :-1]], axis=1)


def _dp_kernel(w_ref, skip_ref, len_ref, selb_ref, sela_ref, out_ref, aB, aL, *, tb):
    i = pl.program_id(0)
    nt = pl.num_programs(0)
    b = skip_ref.shape[0]
    lane = jax.lax.broadcasted_iota(jnp.int32, (b, 128), 1)
    is_lab = lane < 64

    @pl.when(i == 0)
    def _init():
        w0 = w_ref[0]
        wb0 = jnp.where(is_lab, _rot64(w0), w0)
        aB[...] = jnp.where(lane == 0, wb0, NEGBIG)
        aL[...] = jnp.where(lane == 0, w0, NEGBIG)

    skip = skip_ref[...] > 0
    leni = len_ref[...]
    aBv = aB[...]
    aLv = aL[...]
    for tt in range(tb):
        t = i * tb + tt
        w = w_ref[tt]
        wl = jnp.where(is_lab, w, NEGBIG)
        wb = jnp.where(is_lab, _rot64(w), w)
        sh = _shift1(aLv, NEGBIG)
        shs = jnp.where(skip, sh, NEGBIG)
        m3 = jnp.maximum(jnp.maximum(aLv, aBv), shs)
        s3 = jnp.exp(aLv - m3) + jnp.exp(aBv - m3) + jnp.exp(shs - m3)
        newL = m3 + jnp.log(s3) + wl
        m2 = jnp.maximum(aBv, sh)
        s2 = jnp.exp(aBv - m2) + jnp.exp(sh - m2)
        newB = m2 + jnp.log(s2) + wb
        act = (t < leni) & (t > 0)
        aLv = jnp.where(act, jnp.maximum(newL, NEGBIG), aLv)
        aBv = jnp.where(act, jnp.maximum(newB, NEGBIG), aBv)
    aB[...] = aBv
    aL[...] = aLv

    @pl.when(i == nt - 1)
    def _fin():
        ab = jnp.max(jnp.where(selb_ref[...] > 0, aBv, NEGBIG), axis=1,
                     keepdims=True)
        al = jnp.max(jnp.where(sela_ref[...] > 0, aLv, NEGBIG), axis=1,
                     keepdims=True)
        mf = jnp.maximum(ab, al)
        tot = mf + jnp.log(jnp.exp(ab - mf) + jnp.exp(al - mf))
        out_ref[...] = jnp.broadcast_to(-tot, (b, 128))


@jax.jit
def kernel(log_probs, targets, input_lengths, target_lengths):
    B, T, C = log_probs.shape
    L = targets.shape[1]
    targets = targets.astype(jnp.int32)
    input_lengths = input_lengths.astype(jnp.int32)
    target_lengths = target_lengths.astype(jnp.int32)

    cls = jnp.concatenate(
        [targets, jnp.zeros((B, 128 - L), jnp.int32)], axis=1)
    cls = cls[:, None, :]
    lane = jnp.arange(128, dtype=jnp.int32)[None, :]
    prev = jnp.pad(targets, ((0, 0), (1, 0)))[:, :L]
    skip = (lane[:, :L] >= 1) & (targets != prev)
    skipf = jnp.pad(skip.astype(jnp.float32), ((0, 0), (0, 128 - L)))
    lenb = jnp.broadcast_to(input_lengths[:, None], (B, 128))
    selb = (lane == target_lengths[:, None]).astype(jnp.float32)
    sela = (lane == target_lengths[:, None] - 1).astype(jnp.float32)

    w = pl.pallas_call(
        _gather_kernel,
        grid=(B,),
        in_specs=[
            pl.BlockSpec((1, T, C), lambda i: (i, 0, 0)),
            pl.BlockSpec((1, 1, 128), lambda i: (i, 0, 0)),
        ],
        out_specs=pl.BlockSpec((1, T, 128), lambda i: (i, 0, 0)),
        out_shape=jax.ShapeDtypeStruct((B, T, 128), jnp.float32),
        compiler_params=pltpu.CompilerParams(
            dimension_semantics=("arbitrary",)),
    )(log_probs, cls)

    wt = jnp.transpose(w, (1, 0, 2))

    TB = 64
    NT = T // TB
    out = pl.pallas_call(
        functools.partial(_dp_kernel, tb=TB),
        grid=(NT,),
        in_specs=[
            pl.BlockSpec((TB, B, 128), lambda i: (i, 0, 0)),
            pl.BlockSpec((B, 128), lambda i: (0, 0)),
            pl.BlockSpec((B, 128), lambda i: (0, 0)),
            pl.BlockSpec((B, 128), lambda i: (0, 0)),
            pl.BlockSpec((B, 128), lambda i: (0, 0)),
        ],
        out_specs=pl.BlockSpec((B, 128), lambda i: (0, 0)),
        out_shape=jax.ShapeDtypeStruct((B, 128), jnp.float32),
        scratch_shapes=[
            pltpu.VMEM((B, 128), jnp.float32),
            pltpu.VMEM((B, 128), jnp.float32),
        ],
        compiler_params=pltpu.CompilerParams(
            dimension_semantics=("arbitrary",)),
    )(wt, skipf, lenb, selb, sela)

    return out[:, 0]
